# Initial kernel scaffold; baseline (speedup 1.0000x reference)
import jax
import jax.numpy as jnp
from jax.experimental import pallas as pl


def kernel(xyz):
    raise NotImplementedError("write your pallas kernel here")


# speedup vs baseline: 6.8312x; 6.8312x over previous
import functools

import jax
import jax.numpy as jnp
from jax import lax
from jax.experimental import pallas as pl
from jax.experimental.pallas import tpu as pltpu
from jax.experimental.pallas import tpu_sc as plsc

B = 16
N = 8192
G = 512
K = 32
GC = 128

_F32 = jnp.float32
_I32 = jnp.int32


def _fps_body(x_ref, y_ref, z_ref, cx_ref, cy_ref, cz_ref):
    x = x_ref[...]
    y = y_ref[...]
    z = z_ref[...]
    iota_n = lax.broadcasted_iota(_I32, (B, N), 1)
    iota_g = lax.broadcasted_iota(_I32, (B, G), 1)

    def step(i, carry):
        dist, far, cxa, cya, cza = carry
        oh = iota_n == far
        cx = jnp.sum(jnp.where(oh, x, 0.0), axis=1, keepdims=True)
        cy = jnp.sum(jnp.where(oh, y, 0.0), axis=1, keepdims=True)
        cz = jnp.sum(jnp.where(oh, z, 0.0), axis=1, keepdims=True)
        cxa = jnp.where(iota_g == i, cx, cxa)
        cya = jnp.where(iota_g == i, cy, cya)
        cza = jnp.where(iota_g == i, cz, cza)
        dx = x - cx
        dy = y - cy
        dz = z - cz
        d = (dx * dx + dy * dy) + dz * dz
        dist = jnp.minimum(dist, d)
        m = jnp.max(dist, axis=1, keepdims=True)
        far = jnp.min(jnp.where(dist == m, iota_n, N), axis=1, keepdims=True)
        return dist, far, cxa, cya, cza

    init = (
        jnp.full((B, N), 1e10, _F32),
        jnp.zeros((B, 1), _I32),
        jnp.zeros((B, G), _F32),
        jnp.zeros((B, G), _F32),
        jnp.zeros((B, G), _F32),
    )
    _, _, cxa, cya, cza = lax.fori_loop(0, G, step, init)
    cx_ref[...] = cxa
    cy_ref[...] = cya
    cz_ref[...] = cza


def _fps(x2, y2, z2):
    out = jax.ShapeDtypeStruct((B, G), _F32)
    return pl.pallas_call(
        _fps_body,
        out_shape=(out, out, out),
    )(x2, y2, z2)


def _knn_body(x_ref, y_ref, z_ref, qx_ref, qy_ref, qz_ref, idx_ref):
    x = x_ref[0]
    y = y_ref[0]
    z = z_ref[0]
    qx = qx_ref[0]
    qy = qy_ref[0]
    qz = qz_ref[0]
    r2 = (x * x + y * y) + z * z
    q2 = (qx * qx + qy * qy) + qz * qz
    cross = (qx * x + qy * y) + qz * z
    d2 = (q2 + r2) - 2.0 * cross

    iota_n = lax.broadcasted_iota(_I32, (GC, N), 1)
    iota_k = lax.broadcasted_iota(_I32, (GC, K), 1)
    inf = jnp.float32(jnp.inf)

    def ext(k, carry):
        d2c, acc = carry
        m = jnp.min(d2c, axis=1, keepdims=True)
        am = jnp.min(jnp.where(d2c == m, iota_n, N), axis=1, keepdims=True)
        acc = jnp.where(iota_k == k, am, acc)
        d2c = jnp.where(iota_n == am, inf, d2c)
        return d2c, acc

    _, acc = lax.fori_loop(0, K, ext, (d2, jnp.zeros((GC, K), _I32)))
    idx_ref[0] = acc


def _knn(x3, y3, z3, qx3, qy3, qz3):
    pts_spec = pl.BlockSpec((1, 1, N), lambda b, g: (b, 0, 0))
    q_spec = pl.BlockSpec((1, GC, 1), lambda b, g: (b, g, 0))
    out_spec = pl.BlockSpec((1, GC, K), lambda b, g: (b, g, 0))
    return pl.pallas_call(
        _knn_body,
        grid=(B, G // GC),
        in_specs=[pts_spec, pts_spec, pts_spec, q_spec, q_spec, q_spec],
        out_specs=out_spec,
        out_shape=jax.ShapeDtypeStruct((B, G, K), _I32),
    )(x3, y3, z3, qx3, qy3, qz3)


def _sc_gather(xyz4, nidx2):
    info = plsc.get_sparse_core_info()
    nc, ns = info.num_cores, info.num_subcores
    nw = nc * ns
    tot = B * G * K
    rpt = tot // nw
    jpt = rpt // 128
    mesh = plsc.VectorSubcoreMesh(core_axis_name="c", subcore_axis_name="s")

    @functools.partial(
        pl.kernel,
        mesh=mesh,
        compiler_params=pltpu.CompilerParams(use_tc_tiling_on_sc=False),
        out_type=jax.ShapeDtypeStruct((tot, 4), _F32),
        scratch_types=[
            pltpu.VMEM((jpt, 128), _I32),
            pltpu.VMEM((rpt, 4), _F32),
            pltpu.SemaphoreType.DMA,
        ],
    )
    def k(xyz4_hbm, nidx_hbm, out_hbm, idx_v, rows_v, sem):
        wid = lax.axis_index("s") * nc + lax.axis_index("c")
        base_r = wid * rpt
        pltpu.sync_copy(nidx_hbm.at[pl.ds(wid * jpt, jpt)], idx_v)

        def gather_chunk(j, _):
            pltpu.async_copy(
                xyz4_hbm.at[idx_v.at[j]],
                rows_v.at[pl.ds(j * 128, 128)],
                sem,
            ).wait()
            return 0

        lax.fori_loop(0, jpt, gather_chunk, 0)
        pltpu.sync_copy(rows_v, out_hbm.at[pl.ds(base_r, rpt)])

    return k(xyz4, nidx2)


_SUBR = 1024


def _sub_body(a_ref, c_ref, o_ref):
    o_ref[...] = a_ref[...] - c_ref[...]


def _subtract(rows128, cen128):
    spec = pl.BlockSpec((_SUBR, K * 4), lambda i: (i, 0))
    return pl.pallas_call(
        _sub_body,
        grid=((B * G) // _SUBR,),
        in_specs=[spec, spec],
        out_specs=spec,
        out_shape=jax.ShapeDtypeStruct((B * G, K * 4), _F32),
    )(rows128, cen128)


def kernel(xyz):
    xyz = xyz.astype(_F32)
    x2 = xyz[:, :, 0]
    y2 = xyz[:, :, 1]
    z2 = xyz[:, :, 2]
    cx, cy, cz = _fps(x2, y2, z2)

    idx = _knn(
        x2[:, None, :], y2[:, None, :], z2[:, None, :],
        cx[:, :, None], cy[:, :, None], cz[:, :, None],
    )

    flat_idx = (idx + (jnp.arange(B, dtype=_I32) * N)[:, None, None]).reshape(-1)
    nidx2 = flat_idx.reshape(-1, 128)
    xyz4 = jnp.concatenate(
        [xyz.reshape(B * N, 3), jnp.zeros((B * N, 1), _F32)], axis=1
    )

    out4 = _sc_gather(xyz4, nidx2)
    cen4 = jnp.stack([cx, cy, cz, jnp.zeros_like(cx)], axis=-1).reshape(B * G, 4)
    cen128 = jnp.tile(cen4, (1, K))
    nb128 = _subtract(out4.reshape(B * G, K * 4), cen128)
    neighborhood = nb128.reshape(B, G, K, 4)[..., :3]
    center = jnp.stack([cx, cy, cz], axis=-1)
    return (neighborhood, center)


# speedup vs baseline: 12.0921x; 1.7701x over previous
import functools

import jax
import jax.numpy as jnp
from jax import lax
from jax.experimental import pallas as pl
from jax.experimental.pallas import tpu as pltpu
from jax.experimental.pallas import tpu_sc as plsc

B = 16
N = 8192
G = 512
K = 32
GC = 128

_F32 = jnp.float32
_I32 = jnp.int32


def _fps_body(x_ref, y_ref, z_ref, cx_ref, cy_ref, cz_ref):
    x = x_ref[...]
    y = y_ref[...]
    z = z_ref[...]
    iota_n = lax.broadcasted_iota(_I32, (B, N), 1)
    iota_g = lax.broadcasted_iota(_I32, (B, G), 1)

    def step(i, carry):
        dist, far, cxa, cya, cza = carry
        oh = iota_n == far
        cx = jnp.sum(jnp.where(oh, x, 0.0), axis=1, keepdims=True)
        cy = jnp.sum(jnp.where(oh, y, 0.0), axis=1, keepdims=True)
        cz = jnp.sum(jnp.where(oh, z, 0.0), axis=1, keepdims=True)
        cxa = jnp.where(iota_g == i, cx, cxa)
        cya = jnp.where(iota_g == i, cy, cya)
        cza = jnp.where(iota_g == i, cz, cza)
        dx = x - cx
        dy = y - cy
        dz = z - cz
        d = (dx * dx + dy * dy) + dz * dz
        dist = jnp.minimum(dist, d)
        m = jnp.max(dist, axis=1, keepdims=True)
        far = jnp.min(jnp.where(dist == m, iota_n, N), axis=1, keepdims=True)
        return dist, far, cxa, cya, cza

    init = (
        jnp.full((B, N), 1e10, _F32),
        jnp.zeros((B, 1), _I32),
        jnp.zeros((B, G), _F32),
        jnp.zeros((B, G), _F32),
        jnp.zeros((B, G), _F32),
    )
    _, _, cxa, cya, cza = lax.fori_loop(0, G, step, init)
    cx_ref[...] = cxa
    cy_ref[...] = cya
    cz_ref[...] = cza


def _fps(x2, y2, z2):
    out = jax.ShapeDtypeStruct((B, G), _F32)
    return pl.pallas_call(
        _fps_body,
        out_shape=(out, out, out),
    )(x2, y2, z2)


def _knn_body(x_ref, y_ref, z_ref, qx_ref, qy_ref, qz_ref, d2_ref, tub_ref):
    x = x_ref[0]
    y = y_ref[0]
    z = z_ref[0]
    qx = qx_ref[0]
    qy = qy_ref[0]
    qz = qz_ref[0]
    r2 = (x * x + y * y) + z * z
    q2 = (qx * qx + qy * qy) + qz * qz
    cross = (qx * x + qy * y) + qz * z
    d2 = (q2 + r2) - 2.0 * cross
    d2_ref[0] = d2

    gw = N // K
    tub = jnp.min(d2[:, 0:gw], axis=1, keepdims=True)
    for j in range(1, K):
        gm = jnp.min(d2[:, j * gw:(j + 1) * gw], axis=1, keepdims=True)
        tub = jnp.maximum(tub, gm)
    tub_ref[0] = tub


def _knn(x3, y3, z3, qx3, qy3, qz3):
    pts_spec = pl.BlockSpec((1, 1, N), lambda b, g: (b, 0, 0))
    q_spec = pl.BlockSpec((1, GC, 1), lambda b, g: (b, g, 0))
    return pl.pallas_call(
        _knn_body,
        grid=(B, G // GC),
        in_specs=[pts_spec, pts_spec, pts_spec, q_spec, q_spec, q_spec],
        out_specs=[
            pl.BlockSpec((1, GC, N), lambda b, g: (b, g, 0)),
            pl.BlockSpec((1, GC, 1), lambda b, g: (b, g, 0)),
        ],
        out_shape=[
            jax.ShapeDtypeStruct((B, G, N), _F32),
            jax.ShapeDtypeStruct((B, G, 1), _F32),
        ],
    )(x3, y3, z3, qx3, qy3, qz3)


def _merge32(t0, t1, u0, u1, ck, cv):
    ck, cv = plsc.sort_key_val(ck, cv)
    rck = lax.rev(ck, (0,))
    rcv = lax.rev(cv, (0,))
    keep = t1 <= rck
    n1k = jnp.where(keep, t1, rck)
    n1v = jnp.where(keep, u1, rcv)
    lo = t0 <= n1k
    lok = jnp.where(lo, t0, n1k)
    lov = jnp.where(lo, u0, n1v)
    hik = jnp.where(lo, n1k, t0)
    hiv = jnp.where(lo, n1v, u0)
    t0, u0 = plsc.sort_key_val(lok, lov)
    t1, u1 = plsc.sort_key_val(hik, hiv)
    return t0, t1, u0, u1


def _sc_topk(d2f, tub16):
    info = plsc.get_sparse_core_info()
    nc, ns = info.num_cores, info.num_subcores
    nw = nc * ns
    rows = B * G
    rpt = rows // nw
    rg = 4
    ng = rpt // rg
    mesh = plsc.VectorSubcoreMesh(core_axis_name="c", subcore_axis_name="s")
    inf = jnp.float32(jnp.inf)

    @functools.partial(
        pl.kernel,
        mesh=mesh,
        compiler_params=pltpu.CompilerParams(
            use_tc_tiling_on_sc=False, needs_layout_passes=False
        ),
        out_type=jax.ShapeDtypeStruct((rows, K), _I32),
        scratch_types=[
            pltpu.VMEM((rg, N), _F32),
            pltpu.VMEM((rg, N), _F32),
            pltpu.VMEM((N + 16,), _F32),
            pltpu.VMEM((N + 16,), _I32),
            pltpu.VMEM((rpt, 16), _F32),
            pltpu.VMEM((rpt, K), _I32),
            pltpu.SemaphoreType.DMA,
            pltpu.SemaphoreType.DMA,
        ],
    )
    def k(d2_hbm, tub_hbm, out_hbm, da, db, vals, vidx, tubv, outv, sema, semb):
        wid = lax.axis_index("s") * nc + lax.axis_index("c")
        base = wid * rpt
        lane = lax.iota(_I32, 16)
        pltpu.sync_copy(tub_hbm.at[pl.ds(base, rpt)], tubv)

        def do_row(row_ref, r_local, thr):
            def sweep(sb, cnt):
                for cc in range(8):
                    off = sb * 128 + cc * 16
                    v = row_ref[pl.ds(off, 16)]
                    mask = v <= thr
                    keym = jnp.where(mask, v, inf)
                    ks, vs = plsc.sort_key_val(keym, off + lane)
                    vals[pl.ds(cnt, 16)] = ks
                    vidx[pl.ds(cnt, 16)] = vs
                    pc = plsc.all_reduce_population_count(mask)
                    cnt = cnt + pc[0]
                return cnt

            cnt = lax.fori_loop(0, N // 128, sweep, jnp.int32(0))
            vals[pl.ds(cnt, 16)] = jnp.full((16,), inf, _F32)

            def mrg(j, carry):
                t0, t1, u0, u1 = carry
                ck = vals[pl.ds(j * 16, 16)]
                cv = vidx[pl.ds(j * 16, 16)]
                return _merge32(t0, t1, u0, u1, ck, cv)

            init = (
                jnp.full((16,), inf, _F32),
                jnp.full((16,), inf, _F32),
                jnp.zeros((16,), _I32),
                jnp.zeros((16,), _I32),
            )
            nmerge = lax.shift_right_logical(cnt + 15, 4)
            _, _, u0, u1 = lax.fori_loop(0, nmerge, mrg, init)
            outv[r_local, pl.ds(0, 16)] = u0
            outv[r_local, pl.ds(16, 16)] = u1

        def thr_of(i):
            return tubv[i, :]

        def group(g, _):
            pltpu.async_copy(d2_hbm.at[pl.ds(base + g * rg, rg)], da, sema).wait()
            for rl in range(rg):
                i = g * rg + rl
                do_row(da.at[rl], i, thr_of(i))
            return 0

        lax.fori_loop(0, ng, group, 0)
        pltpu.sync_copy(outv, out_hbm.at[pl.ds(base, rpt)])

    return k(d2f, tub16)


def _sc_gather(xyz4, nidx2):
    info = plsc.get_sparse_core_info()
    nc, ns = info.num_cores, info.num_subcores
    nw = nc * ns
    tot = B * G * K
    rpt = tot // nw
    jpt = rpt // 128
    mesh = plsc.VectorSubcoreMesh(core_axis_name="c", subcore_axis_name="s")

    @functools.partial(
        pl.kernel,
        mesh=mesh,
        compiler_params=pltpu.CompilerParams(use_tc_tiling_on_sc=False),
        out_type=jax.ShapeDtypeStruct((tot, 4), _F32),
        scratch_types=[
            pltpu.VMEM((jpt, 128), _I32),
            pltpu.VMEM((rpt, 4), _F32),
            pltpu.SemaphoreType.DMA,
        ],
    )
    def k(xyz4_hbm, nidx_hbm, out_hbm, idx_v, rows_v, sem):
        wid = lax.axis_index("s") * nc + lax.axis_index("c")
        base_r = wid * rpt
        pltpu.sync_copy(nidx_hbm.at[pl.ds(wid * jpt, jpt)], idx_v)

        def gather_chunk(j, _):
            pltpu.async_copy(
                xyz4_hbm.at[idx_v.at[j]],
                rows_v.at[pl.ds(j * 128, 128)],
                sem,
            ).wait()
            return 0

        lax.fori_loop(0, jpt, gather_chunk, 0)
        pltpu.sync_copy(rows_v, out_hbm.at[pl.ds(base_r, rpt)])

    return k(xyz4, nidx2)


_SUBR = 1024


def _sub_body(a_ref, c_ref, o_ref):
    o_ref[...] = a_ref[...] - c_ref[...]


def _subtract(rows128, cen128):
    spec = pl.BlockSpec((_SUBR, K * 4), lambda i: (i, 0))
    return pl.pallas_call(
        _sub_body,
        grid=((B * G) // _SUBR,),
        in_specs=[spec, spec],
        out_specs=spec,
        out_shape=jax.ShapeDtypeStruct((B * G, K * 4), _F32),
    )(rows128, cen128)


def kernel(xyz):
    xyz = xyz.astype(_F32)
    x2 = xyz[:, :, 0]
    y2 = xyz[:, :, 1]
    z2 = xyz[:, :, 2]
    cx, cy, cz = _fps(x2, y2, z2)

    d2, tub = _knn(
        x2[:, None, :], y2[:, None, :], z2[:, None, :],
        cx[:, :, None], cy[:, :, None], cz[:, :, None],
    )
    tub16 = jnp.tile(tub.reshape(B * G, 1), (1, 16))
    idx = _sc_topk(d2.reshape(B * G, N), tub16).reshape(B, G, K)

    flat_idx = (idx + (jnp.arange(B, dtype=_I32) * N)[:, None, None]).reshape(-1)
    nidx2 = flat_idx.reshape(-1, 128)
    xyz4 = jnp.concatenate(
        [xyz.reshape(B * N, 3), jnp.zeros((B * N, 1), _F32)], axis=1
    )

    out4 = _sc_gather(xyz4, nidx2)
    cen4 = jnp.stack([cx, cy, cz, jnp.zeros_like(cx)], axis=-1).reshape(B * G, 4)
    cen128 = jnp.tile(cen4, (1, K))
    nb128 = _subtract(out4.reshape(B * G, K * 4), cen128)
    neighborhood = nb128.reshape(B, G, K, 4)[..., :3]
    center = jnp.stack([cx, cy, cz], axis=-1)
    return (neighborhood, center)


# speedup vs baseline: 22.6672x; 1.8745x over previous
import functools

import jax
import jax.numpy as jnp
from jax import lax
from jax.experimental import pallas as pl
from jax.experimental.pallas import tpu as pltpu
from jax.experimental.pallas import tpu_sc as plsc

B = 16
N = 8192
G = 512
K = 32
GC = 128

_F32 = jnp.float32
_I32 = jnp.int32


def _fps_body(x_ref, y_ref, z_ref, cx_ref, cy_ref, cz_ref):
    x = x_ref[...]
    y = y_ref[...]
    z = z_ref[...]
    iota_n = lax.broadcasted_iota(_I32, (B, N), 1)
    iota_g = lax.broadcasted_iota(_I32, (B, G), 1)

    def step(i, carry):
        dist, far, cxa, cya, cza = carry
        oh = iota_n == far
        cx = jnp.sum(jnp.where(oh, x, 0.0), axis=1, keepdims=True)
        cy = jnp.sum(jnp.where(oh, y, 0.0), axis=1, keepdims=True)
        cz = jnp.sum(jnp.where(oh, z, 0.0), axis=1, keepdims=True)
        cxa = jnp.where(iota_g == i, cx, cxa)
        cya = jnp.where(iota_g == i, cy, cya)
        cza = jnp.where(iota_g == i, cz, cza)
        dx = x - cx
        dy = y - cy
        dz = z - cz
        d = (dx * dx + dy * dy) + dz * dz
        dist = jnp.minimum(dist, d)
        m = jnp.max(dist, axis=1, keepdims=True)
        far = jnp.min(jnp.where(dist == m, iota_n, N), axis=1, keepdims=True)
        return dist, far, cxa, cya, cza

    init = (
        jnp.full((B, N), 1e10, _F32),
        jnp.zeros((B, 1), _I32),
        jnp.zeros((B, G), _F32),
        jnp.zeros((B, G), _F32),
        jnp.zeros((B, G), _F32),
    )
    _, _, cxa, cya, cza = lax.fori_loop(0, G, step, init)
    cx_ref[...] = cxa
    cy_ref[...] = cya
    cz_ref[...] = cza


def _fps(x2, y2, z2):
    out = jax.ShapeDtypeStruct((B, G), _F32)
    return pl.pallas_call(
        _fps_body,
        out_shape=(out, out, out),
    )(x2, y2, z2)


def _knn_body(x_ref, y_ref, z_ref, qx_ref, qy_ref, qz_ref, d2_ref, tub_ref):
    x = x_ref[0]
    y = y_ref[0]
    z = z_ref[0]
    qx = qx_ref[0]
    qy = qy_ref[0]
    qz = qz_ref[0]
    r2 = (x * x + y * y) + z * z
    q2 = (qx * qx + qy * qy) + qz * qz
    cross = (qx * x + qy * y) + qz * z
    d2 = (q2 + r2) - 2.0 * cross
    d2_ref[0] = d2

    gw = N // K
    tub = jnp.min(d2[:, 0:gw], axis=1, keepdims=True)
    for j in range(1, K):
        gm = jnp.min(d2[:, j * gw:(j + 1) * gw], axis=1, keepdims=True)
        tub = jnp.maximum(tub, gm)
    tub_ref[0] = tub


def _knn(x3, y3, z3, qx3, qy3, qz3):
    pts_spec = pl.BlockSpec((1, 1, N), lambda b, g: (b, 0, 0))
    q_spec = pl.BlockSpec((1, GC, 1), lambda b, g: (b, g, 0))
    return pl.pallas_call(
        _knn_body,
        grid=(B, G // GC),
        in_specs=[pts_spec, pts_spec, pts_spec, q_spec, q_spec, q_spec],
        out_specs=[
            pl.BlockSpec((1, GC, N), lambda b, g: (b, g, 0)),
            pl.BlockSpec((1, GC, 1), lambda b, g: (b, g, 0)),
        ],
        out_shape=[
            jax.ShapeDtypeStruct((B, G, N), _F32),
            jax.ShapeDtypeStruct((B, G, 1), _F32),
        ],
    )(x3, y3, z3, qx3, qy3, qz3)


def _merge32(t0, t1, u0, u1, ck, cv):
    ck, cv = plsc.sort_key_val(ck, cv)
    rck = lax.rev(ck, (0,))
    rcv = lax.rev(cv, (0,))
    keep = t1 <= rck
    n1k = jnp.where(keep, t1, rck)
    n1v = jnp.where(keep, u1, rcv)
    lo = t0 <= n1k
    lok = jnp.where(lo, t0, n1k)
    lov = jnp.where(lo, u0, n1v)
    hik = jnp.where(lo, n1k, t0)
    hiv = jnp.where(lo, n1v, u0)
    t0, u0 = plsc.sort_key_val(lok, lov)
    t1, u1 = plsc.sort_key_val(hik, hiv)
    return t0, t1, u0, u1


def _sc_topk(d2f, tub16):
    info = plsc.get_sparse_core_info()
    nc, ns = info.num_cores, info.num_subcores
    nw = nc * ns
    rows = B * G
    rpt = rows // nw
    rg = 4
    ng = rpt // rg
    mesh = plsc.VectorSubcoreMesh(core_axis_name="c", subcore_axis_name="s")
    inf = jnp.float32(jnp.inf)

    @functools.partial(
        pl.kernel,
        mesh=mesh,
        compiler_params=pltpu.CompilerParams(
            use_tc_tiling_on_sc=False, needs_layout_passes=False
        ),
        out_type=jax.ShapeDtypeStruct((rows, K), _I32),
        scratch_types=[
            pltpu.VMEM((rg, N), _F32),
            pltpu.VMEM((rg, N), _F32),
            pltpu.VMEM((N + 16,), _F32),
            pltpu.VMEM((N + 16,), _I32),
            pltpu.VMEM((rpt, 16), _F32),
            pltpu.VMEM((rpt, K), _I32),
            pltpu.SemaphoreType.DMA,
            pltpu.SemaphoreType.DMA,
        ],
    )
    def k(d2_hbm, tub_hbm, out_hbm, da, db, vals, vidx, tubv, outv, sema, semb):
        wid = lax.axis_index("s") * nc + lax.axis_index("c")
        base = wid * rpt
        lane = lax.iota(_I32, 16)
        pltpu.sync_copy(tub_hbm.at[pl.ds(base, rpt)], tubv)

        def do_row(row_ref, r_local, thr):
            def sweep(sb, cnt):
                stash = []
                for cc in range(8):
                    off = sb * 128 + cc * 16
                    v = row_ref[pl.ds(off, 16)]
                    mask = v <= thr
                    pc = plsc.all_reduce_population_count(mask)
                    keym = jnp.where(mask, v, inf)
                    ks, vs = plsc.sort_key_val(keym, off + lane)
                    stash.append((ks, vs, pc[0]))
                for ks, vs, sc in stash:
                    vals[pl.ds(cnt, 16)] = ks
                    vidx[pl.ds(cnt, 16)] = vs
                    cnt = cnt + sc
                return cnt

            cnt = lax.fori_loop(0, N // 128, sweep, jnp.int32(0))
            vals[pl.ds(cnt, 16)] = jnp.full((16,), inf, _F32)

            def mrg(j, carry):
                t0, t1, u0, u1 = carry
                ck = vals[pl.ds(j * 16, 16)]
                cv = vidx[pl.ds(j * 16, 16)]
                return _merge32(t0, t1, u0, u1, ck, cv)

            init = (
                jnp.full((16,), inf, _F32),
                jnp.full((16,), inf, _F32),
                jnp.zeros((16,), _I32),
                jnp.zeros((16,), _I32),
            )
            nmerge = lax.shift_right_logical(cnt + 15, 4)
            _, _, u0, u1 = lax.fori_loop(0, nmerge, mrg, init)
            outv[r_local, pl.ds(0, 16)] = u0
            outv[r_local, pl.ds(16, 16)] = u1

        def thr_of(i):
            return tubv[i, :]

        def group(g, _):
            pltpu.async_copy(d2_hbm.at[pl.ds(base + g * rg, rg)], da, sema).wait()
            for rl in range(rg):
                i = g * rg + rl
                do_row(da.at[rl], i, thr_of(i))
            return 0

        lax.fori_loop(0, ng, group, 0)
        pltpu.sync_copy(outv, out_hbm.at[pl.ds(base, rpt)])

    return k(d2f, tub16)


def _sc_gather(xyz4, nidx2):
    info = plsc.get_sparse_core_info()
    nc, ns = info.num_cores, info.num_subcores
    nw = nc * ns
    tot = B * G * K
    rpt = tot // nw
    jpt = rpt // 128
    mesh = plsc.VectorSubcoreMesh(core_axis_name="c", subcore_axis_name="s")

    @functools.partial(
        pl.kernel,
        mesh=mesh,
        compiler_params=pltpu.CompilerParams(use_tc_tiling_on_sc=False),
        out_type=jax.ShapeDtypeStruct((tot, 4), _F32),
        scratch_types=[
            pltpu.VMEM((jpt, 128), _I32),
            pltpu.VMEM((rpt, 4), _F32),
            pltpu.SemaphoreType.DMA,
        ],
    )
    def k(xyz4_hbm, nidx_hbm, out_hbm, idx_v, rows_v, sem):
        wid = lax.axis_index("s") * nc + lax.axis_index("c")
        base_r = wid * rpt
        pltpu.sync_copy(nidx_hbm.at[pl.ds(wid * jpt, jpt)], idx_v)

        def gather_chunk(j, _):
            pltpu.async_copy(
                xyz4_hbm.at[idx_v.at[j]],
                rows_v.at[pl.ds(j * 128, 128)],
                sem,
            ).wait()
            return 0

        lax.fori_loop(0, jpt, gather_chunk, 0)
        pltpu.sync_copy(rows_v, out_hbm.at[pl.ds(base_r, rpt)])

    return k(xyz4, nidx2)


_SUBR = 1024


def _sub_body(a_ref, c_ref, o_ref):
    o_ref[...] = a_ref[...] - c_ref[...]


def _subtract(rows128, cen128):
    spec = pl.BlockSpec((_SUBR, K * 4), lambda i: (i, 0))
    return pl.pallas_call(
        _sub_body,
        grid=((B * G) // _SUBR,),
        in_specs=[spec, spec],
        out_specs=spec,
        out_shape=jax.ShapeDtypeStruct((B * G, K * 4), _F32),
    )(rows128, cen128)


def kernel(xyz):
    xyz = xyz.astype(_F32)
    x2 = xyz[:, :, 0]
    y2 = xyz[:, :, 1]
    z2 = xyz[:, :, 2]
    cx, cy, cz = _fps(x2, y2, z2)

    d2, tub = _knn(
        x2[:, None, :], y2[:, None, :], z2[:, None, :],
        cx[:, :, None], cy[:, :, None], cz[:, :, None],
    )
    tub16 = jnp.tile(tub.reshape(B * G, 1), (1, 16))
    idx = _sc_topk(d2.reshape(B * G, N), tub16).reshape(B, G, K)

    flat_idx = (idx + (jnp.arange(B, dtype=_I32) * N)[:, None, None]).reshape(-1)
    nidx2 = flat_idx.reshape(-1, 128)
    xyz4 = jnp.concatenate(
        [xyz.reshape(B * N, 3), jnp.zeros((B * N, 1), _F32)], axis=1
    )

    out4 = _sc_gather(xyz4, nidx2)
    cen4 = jnp.stack([cx, cy, cz, jnp.zeros_like(cx)], axis=-1).reshape(B * G, 4)
    cen128 = jnp.tile(cen4, (1, K))
    nb128 = _subtract(out4.reshape(B * G, K * 4), cen128)
    neighborhood = nb128.reshape(B, G, K, 4)[..., :3]
    center = jnp.stack([cx, cy, cz], axis=-1)
    return (neighborhood, center)
